# Initial kernel scaffold; baseline (speedup 1.0000x reference)
import jax
import jax.numpy as jnp
from jax.experimental import pallas as pl


def kernel(x, embed, W, b):
    raise NotImplementedError("write your pallas kernel here")


# speedup vs baseline: 1.8150x; 1.8150x over previous
import functools

import jax
import jax.numpy as jnp
from jax import lax
from jax.experimental import pallas as pl
from jax.experimental.pallas import tpu as pltpu
from jax.experimental.pallas import tpu_sc as plsc

VOCAB = 256
DIM = 64

NC = 2
NS = 16
NW = NC * NS

CHUNK = 128


def _table_body(embed_ref, w_ref, b_ref, t_ref):
    t_ref[...] = (
        jnp.dot(embed_ref[...], w_ref[...].T, preferred_element_type=jnp.float32)
        + b_ref[...]
    )


def _make_table(embed, W, b):
    return pl.pallas_call(
        _table_body,
        out_shape=jax.ShapeDtypeStruct((VOCAB, VOCAB), jnp.float32),
    )(embed, W, b.reshape(1, VOCAB))


def _make_gather(n_tokens):
    assert n_tokens % (NW * CHUNK) == 0
    bpw = n_tokens // NW
    nchunk = bpw // CHUNK

    mesh = plsc.VectorSubcoreMesh(core_axis_name="c", subcore_axis_name="s")

    @functools.partial(
        pl.kernel,
        mesh=mesh,
        out_type=jax.ShapeDtypeStruct((n_tokens, VOCAB), jnp.float32),
        scratch_types=[
            pltpu.VMEM((nchunk, CHUNK), jnp.int32),
            pltpu.VMEM((CHUNK, VOCAB), jnp.float32),
            pltpu.VMEM((CHUNK, VOCAB), jnp.float32),
            pltpu.SemaphoreType.DMA,
            pltpu.SemaphoreType.DMA,
        ],
    )
    def gather(table_hbm, idx_hbm, out_hbm, idx_v, buf0, buf1, sem0, sem1):
        wid = lax.axis_index("s") * NC + lax.axis_index("c")
        base = wid * bpw
        pltpu.sync_copy(idx_hbm.at[wid], idx_v)
        bufs = (buf0, buf1)
        sems = (sem0, sem1)
        copies = [None, None]
        copies[0] = pltpu.async_copy(table_hbm.at[idx_v.at[0]], buf0, sem0)
        for j in range(nchunk):
            nj = j + 1
            if nj < nchunk:
                copies[nj % 2] = pltpu.async_copy(
                    table_hbm.at[idx_v.at[nj]], bufs[nj % 2], sems[nj % 2]
                )
            copies[j % 2].wait()
            pltpu.sync_copy(
                bufs[j % 2], out_hbm.at[pl.ds(base + j * CHUNK, CHUNK)]
            )

    return gather


def kernel(x, embed, W, b):
    batch, seq = x.shape
    n_tokens = batch * seq
    table = _make_table(embed, W, b)
    idx = x.reshape(NW, n_tokens // (NW * CHUNK), CHUNK)
    out = _make_gather(n_tokens)(table, idx)
    return out.reshape(batch, seq, VOCAB)


# speedup vs baseline: 1.8199x; 1.0027x over previous
import functools

import jax
import jax.numpy as jnp
from jax import lax
from jax.experimental import pallas as pl
from jax.experimental.pallas import tpu as pltpu
from jax.experimental.pallas import tpu_sc as plsc

VOCAB = 256
DIM = 64

NC = 2
NS = 16
NW = NC * NS

CHUNK = 128


def _table_body(embed_ref, w_ref, b_ref, t_ref):
    t_ref[...] = (
        jnp.dot(embed_ref[...], w_ref[...].T, preferred_element_type=jnp.float32)
        + b_ref[...]
    )


def _make_table(embed, W, b):
    return pl.pallas_call(
        _table_body,
        out_shape=jax.ShapeDtypeStruct((VOCAB, VOCAB), jnp.float32),
    )(embed, W, b.reshape(1, VOCAB))


def _make_gather(n_tokens):
    assert n_tokens % (NW * CHUNK) == 0
    bpw = n_tokens // NW
    nchunk = bpw // CHUNK

    mesh = plsc.VectorSubcoreMesh(core_axis_name="c", subcore_axis_name="s")

    nbuf = 3
    assert nchunk >= nbuf

    @functools.partial(
        pl.kernel,
        mesh=mesh,
        out_type=jax.ShapeDtypeStruct((n_tokens, VOCAB), jnp.float32),
        scratch_types=[
            pltpu.VMEM((nchunk, CHUNK), jnp.int32),
        ]
        + [pltpu.VMEM((CHUNK, VOCAB), jnp.float32) for _ in range(nbuf)]
        + [pltpu.SemaphoreType.DMA for _ in range(2 * nbuf)],
    )
    def gather(table_hbm, idx_hbm, out_hbm, idx_v, *rest):
        bufs = rest[:nbuf]
        gsems = rest[nbuf : 2 * nbuf]
        wsems = rest[2 * nbuf :]
        wid = lax.axis_index("s") * NC + lax.axis_index("c")
        base = wid * bpw
        pltpu.sync_copy(idx_hbm.at[wid], idx_v)

        def fire_gather(j):
            return pltpu.async_copy(
                table_hbm.at[idx_v.at[j]], bufs[j % nbuf], gsems[j % nbuf]
            )

        def fire_write(j):
            return pltpu.async_copy(
                bufs[j % nbuf],
                out_hbm.at[pl.ds(base + j * CHUNK, CHUNK)],
                wsems[j % nbuf],
            )

        g = [None] * nchunk
        w = [None] * nchunk
        g[0] = fire_gather(0)
        g[1] = fire_gather(1)
        for j in range(nchunk):
            g[j].wait()
            w[j] = fire_write(j)
            nx = j + 2
            if nx < nchunk:
                if nx >= nbuf:
                    w[nx - nbuf].wait()
                g[nx] = fire_gather(nx)
        for j in range(max(0, nchunk - nbuf), nchunk):
            w[j].wait()

    return gather


def kernel(x, embed, W, b):
    batch, seq = x.shape
    n_tokens = batch * seq
    table = _make_table(embed, W, b)
    idx = x.reshape(NW, n_tokens // (NW * CHUNK), CHUNK)
    out = _make_gather(n_tokens)(table, idx)
    return out.reshape(batch, seq, VOCAB)


# speedup vs baseline: 2.0990x; 1.1533x over previous
import functools

import jax
import jax.numpy as jnp
from jax import lax
from jax.experimental import pallas as pl
from jax.experimental.pallas import tpu as pltpu
from jax.experimental.pallas import tpu_sc as plsc

VOCAB = 256
DIM = 64

NC = 2
NS = 16
NW = NC * NS

CHUNK = 64


REP = NW


def _table_body(embed_ref, w_ref, b_ref, t_ref):
    t_ref[0] = (
        jnp.dot(embed_ref[...], w_ref[...].T, preferred_element_type=jnp.float32)
        + b_ref[...]
    )


def _make_table(embed, W, b):
    return pl.pallas_call(
        _table_body,
        grid=(REP,),
        in_specs=[
            pl.BlockSpec((VOCAB, DIM), lambda i: (0, 0)),
            pl.BlockSpec((VOCAB, DIM), lambda i: (0, 0)),
            pl.BlockSpec((1, VOCAB), lambda i: (0, 0)),
        ],
        out_specs=pl.BlockSpec((1, VOCAB, VOCAB), lambda i: (i, 0, 0)),
        out_shape=jax.ShapeDtypeStruct((REP, VOCAB, VOCAB), jnp.float32),
    )(embed, W, b.reshape(1, VOCAB))


def _make_gather(n_tokens):
    assert n_tokens % (NW * CHUNK) == 0
    bpw = n_tokens // NW
    nchunk = bpw // CHUNK

    mesh = plsc.VectorSubcoreMesh(core_axis_name="c", subcore_axis_name="s")

    nbuf = 3
    assert nchunk >= nbuf

    @functools.partial(
        pl.kernel,
        mesh=mesh,
        out_type=jax.ShapeDtypeStruct((n_tokens, VOCAB), jnp.float32),
        scratch_types=[
            pltpu.VMEM((nchunk, CHUNK), jnp.int32),
        ]
        + [pltpu.VMEM((CHUNK, VOCAB), jnp.float32) for _ in range(nbuf)]
        + [pltpu.SemaphoreType.DMA for _ in range(2 * nbuf)],
    )
    def gather(table_hbm, idx_hbm, out_hbm, idx_v, *rest):
        bufs = rest[:nbuf]
        gsems = rest[nbuf : 2 * nbuf]
        wsems = rest[2 * nbuf :]
        wid = lax.axis_index("s") * NC + lax.axis_index("c")
        base = wid * bpw
        pltpu.sync_copy(idx_hbm.at[wid], idx_v)

        def fire_gather(j):
            return pltpu.async_copy(
                table_hbm.at[wid].at[idx_v.at[j]], bufs[j % nbuf], gsems[j % nbuf]
            )

        def fire_write(j):
            return pltpu.async_copy(
                bufs[j % nbuf],
                out_hbm.at[pl.ds(base + j * CHUNK, CHUNK)],
                wsems[j % nbuf],
            )

        g = [None] * nchunk
        w = [None] * nchunk
        g[0] = fire_gather(0)
        g[1] = fire_gather(1)
        for j in range(nchunk):
            g[j].wait()
            w[j] = fire_write(j)
            nx = j + 2
            if nx < nchunk:
                if nx >= nbuf:
                    w[nx - nbuf].wait()
                g[nx] = fire_gather(nx)
        for j in range(max(0, nchunk - nbuf), nchunk):
            w[j].wait()

    return gather


def kernel(x, embed, W, b):
    batch, seq = x.shape
    n_tokens = batch * seq
    table = _make_table(embed, W, b)
    idx = x.reshape(NW, n_tokens // (NW * CHUNK), CHUNK)
    out = _make_gather(n_tokens)(table, idx)
    return out.reshape(batch, seq, VOCAB)
